# Initial kernel scaffold; baseline (speedup 1.0000x reference)
import jax
import jax.numpy as jnp
from jax.experimental import pallas as pl


def kernel(coords, grid0, grid1, grid2, grid3, W0, b0, W1, b1, W2, b2, W3, b3):
    raise NotImplementedError("write your pallas kernel here")


# speedup vs baseline: 25.2428x; 25.2428x over previous
import functools

import jax
import jax.numpy as jnp
from jax import lax
from jax.experimental import pallas as pl
from jax.experimental.pallas import tpu as pltpu
from jax.experimental.pallas import tpu_sc as plsc

RES = (16, 32, 64, 128)
NLEV = 4
F = 8
N_POINTS = 524288
NC, NS, L = 2, 16, 16
NW = NC * NS
PTS_PER_W = N_POINTS // NW
B = 128
NBLK = PTS_PER_W // B
NSUB = B // L


def _sc_trilerp_feats(coords, g0, g1, g2, g3):
    grids = (g0, g1, g2, g3)

    mesh = plsc.VectorSubcoreMesh(core_axis_name="c", subcore_axis_name="s",
                                  num_cores=NC, num_subcores=NS)

    def body(coords_hbm, g0_hbm, g1_hbm, g2_hbm, g3_hbm, feats_hbm,
             g0_v, coords_v, idx_v, w_v, rows_v, feats_v, sem):
        wid = lax.axis_index("s") * NC + lax.axis_index("c")
        w_base = wid * PTS_PER_W
        pltpu.sync_copy(g0_hbm, g0_v)
        iota = lax.iota(jnp.int32, L)

        @pl.loop(0, NBLK)
        def _blk(g):
            base = w_base + g * B
            pltpu.sync_copy(coords_hbm.at[pl.ds(base * 3, 3 * B)], coords_v)

            @pl.loop(0, NSUB)
            def _idx(j):
                p = j * L
                i3 = p * 3 + iota * 3
                x = plsc.load_gather(coords_v, [i3])
                y = plsc.load_gather(coords_v, [i3 + 1])
                z = plsc.load_gather(coords_v, [i3 + 2])
                for l in range(NLEV):
                    r = RES[l]
                    rm1 = float(r - 1)
                    gx = jnp.clip(x * rm1, 0.0, rm1)
                    gy = jnp.clip(y * rm1, 0.0, rm1)
                    gz = jnp.clip(z * rm1, 0.0, rm1)
                    xi = gx.astype(jnp.int32)
                    yi = gy.astype(jnp.int32)
                    zi = gz.astype(jnp.int32)
                    fx = gx - xi.astype(jnp.float32)
                    fy = gy - yi.astype(jnp.float32)
                    fz = gz - zi.astype(jnp.float32)
                    x1 = jnp.minimum(xi + 1, r - 1)
                    y1 = jnp.minimum(yi + 1, r - 1)
                    z1 = jnp.minimum(zi + 1, r - 1)
                    a00 = (xi * r + yi) * r
                    a01 = (xi * r + y1) * r
                    a10 = (x1 * r + yi) * r
                    a11 = (x1 * r + y1) * r
                    corners = (a00 + zi, a00 + z1, a01 + zi, a01 + z1,
                               a10 + zi, a10 + z1, a11 + zi, a11 + z1)
                    ax, ay, az = 1.0 - fx, 1.0 - fy, 1.0 - fz
                    pxy0, pxy1 = ax * ay, ax * fy
                    pxy2, pxy3 = fx * ay, fx * fy
                    wts = (pxy0 * az, pxy0 * fz, pxy1 * az, pxy1 * fz,
                           pxy2 * az, pxy2 * fz, pxy3 * az, pxy3 * fz)
                    for c in range(8):
                        idx_v[l, c, pl.ds(p, L)] = corners[c]
                        w_v[l, c, pl.ds(p, L)] = wts[c]

            copies = []
            for l, g_hbm in ((1, g1_hbm), (2, g2_hbm), (3, g3_hbm)):
                for c in range(8):
                    copies.append(pltpu.async_copy(
                        g_hbm.at[idx_v.at[l, c]], rows_v.at[l - 1, c], sem))
            for cp in copies:
                cp.wait()

            @pl.loop(0, NSUB)
            def _acc(j):
                p = j * L
                row = p + iota
                for l in range(NLEV):
                    for f in range(F):
                        col = jnp.full((L,), f, jnp.int32)
                        acc = jnp.zeros((L,), jnp.float32)
                        for c in range(8):
                            wv = w_v[l, c, pl.ds(p, L)]
                            if l == 0:
                                ridx = idx_v[0, c, pl.ds(p, L)]
                                v = plsc.load_gather(g0_v, [ridx, col])
                            else:
                                v = plsc.load_gather(rows_v.at[l - 1, c],
                                                     [row, col])
                            acc = acc + wv * v
                        plsc.store_scatter(
                            feats_v, [row, jnp.full((L,), l * F + f, jnp.int32)],
                            acc)

            pltpu.sync_copy(feats_v, feats_hbm.at[pl.ds(base, B)])

    kern = pl.kernel(
        body,
        out_type=jax.ShapeDtypeStruct((N_POINTS, NLEV * F), jnp.float32),
        mesh=mesh,
        compiler_params=pltpu.CompilerParams(needs_layout_passes=False,
                                             use_tc_tiling_on_sc=False),
        scratch_types=[
            pltpu.VMEM((RES[0] ** 3, F), jnp.float32),
            pltpu.VMEM((3 * B,), jnp.float32),
            pltpu.VMEM((NLEV, 8, B), jnp.int32),
            pltpu.VMEM((NLEV, 8, B), jnp.float32),
            pltpu.VMEM((NLEV - 1, 8, B, F), jnp.float32),
            pltpu.VMEM((B, NLEV * F), jnp.float32),
            pltpu.SemaphoreType.DMA,
        ],
    )
    return kern(coords, *grids)


BN = 2048


def _mlp_body(x_ref, w0, b0, w1, b1, w2, b2, w3, b3, o_ref):
    x = x_ref[...]
    h = jnp.maximum(jnp.dot(x, w0[...], preferred_element_type=jnp.float32)
                    + b0[...], 0.0)
    h = jnp.maximum(jnp.dot(h, w1[...], preferred_element_type=jnp.float32)
                    + b1[...], 0.0)
    h = jnp.maximum(jnp.dot(h, w2[...], preferred_element_type=jnp.float32)
                    + b2[...], 0.0)
    o_ref[...] = jnp.dot(h, w3[...], preferred_element_type=jnp.float32) + b3[...]


def _mlp(feats, W0, b0, W1, b1, W2, b2, W3, b3):
    def wspec(shape):
        return pl.BlockSpec(shape, lambda i: (0, 0))

    return pl.pallas_call(
        _mlp_body,
        grid=(N_POINTS // BN,),
        in_specs=[pl.BlockSpec((BN, NLEV * F), lambda i: (i, 0)),
                  wspec((32, 64)), wspec((1, 64)),
                  wspec((64, 64)), wspec((1, 64)),
                  wspec((64, 64)), wspec((1, 64)),
                  wspec((64, 4)), wspec((1, 4))],
        out_specs=pl.BlockSpec((BN, 4), lambda i: (i, 0)),
        out_shape=jax.ShapeDtypeStruct((N_POINTS, 4), jnp.float32),
    )(feats, W0, b0.reshape(1, -1), W1, b1.reshape(1, -1),
      W2, b2.reshape(1, -1), W3, b3.reshape(1, -1))


@jax.jit
def kernel(coords, grid0, grid1, grid2, grid3, W0, b0, W1, b1, W2, b2, W3, b3):
    g0 = grid0.reshape(-1, F)
    g1 = grid1.reshape(-1, F)
    g2 = grid2.reshape(-1, F)
    g3 = grid3.reshape(-1, F)
    feats = _sc_trilerp_feats(coords.reshape(-1), g0, g1, g2, g3)
    return _mlp(feats, W0, b0, W1, b1, W2, b2, W3, b3)


# speedup vs baseline: 29.9938x; 1.1882x over previous
import jax
import jax.numpy as jnp
from jax import lax
from jax.experimental import pallas as pl
from jax.experimental.pallas import tpu as pltpu
from jax.experimental.pallas import tpu_sc as plsc

RES = (16, 32, 64, 128)
NLEV = 4
F = 8
F2 = 2 * F
N_POINTS = 524288
NC, NS, L = 2, 16, 16
NW = NC * NS
PTS_PER_W = N_POINTS // NW
B = 256
H = 128
NH = B // H
NBLK = PTS_PER_W // B
NSUB = B // L


def _sc_trilerp_feats(xs, ys, zs, g0, zd1, zd2, zd3):
    mesh = plsc.VectorSubcoreMesh(core_axis_name="c", subcore_axis_name="s",
                                  num_cores=NC, num_subcores=NS)

    def body(xs_hbm, ys_hbm, zs_hbm, g0_hbm, zd1_hbm, zd2_hbm, zd3_hbm,
             feats_hbm, g0_v, xs_v, ys_v, zs_v, idx0_v, pidx_v, w_v, rows_v,
             feats_v, sem):
        wid = lax.axis_index("s") * NC + lax.axis_index("c")
        w_base = wid * PTS_PER_W
        pltpu.sync_copy(g0_hbm, g0_v)
        iota = lax.iota(jnp.int32, L)

        @pl.loop(0, NBLK)
        def _blk(g):
            base = w_base + g * B
            pltpu.sync_copy(xs_hbm.at[pl.ds(base, B)], xs_v)
            pltpu.sync_copy(ys_hbm.at[pl.ds(base, B)], ys_v)
            pltpu.sync_copy(zs_hbm.at[pl.ds(base, B)], zs_v)

            @pl.loop(0, NH)
            def _idxh(h):
                @pl.loop(0, H // L)
                def _idx(jj):
                    p = h * H + jj * L
                    pp = jj * L
                    x = xs_v[pl.ds(p, L)]
                    y = ys_v[pl.ds(p, L)]
                    z = zs_v[pl.ds(p, L)]
                    for l in range(NLEV):
                        r = RES[l]
                        rm1 = float(r - 1)
                        gx = jnp.clip(x * rm1, 0.0, rm1)
                        gy = jnp.clip(y * rm1, 0.0, rm1)
                        gz = jnp.clip(z * rm1, 0.0, rm1)
                        xi = gx.astype(jnp.int32)
                        yi = gy.astype(jnp.int32)
                        zi = gz.astype(jnp.int32)
                        fx = gx - xi.astype(jnp.float32)
                        fy = gy - yi.astype(jnp.float32)
                        fz = gz - zi.astype(jnp.float32)
                        x1 = jnp.minimum(xi + 1, r - 1)
                        y1 = jnp.minimum(yi + 1, r - 1)
                        a00 = (xi * r + yi) * r
                        a01 = (xi * r + y1) * r
                        a10 = (x1 * r + yi) * r
                        a11 = (x1 * r + y1) * r
                        ax, ay, az = 1.0 - fx, 1.0 - fy, 1.0 - fz
                        pxy0, pxy1 = ax * ay, ax * fy
                        pxy2, pxy3 = fx * ay, fx * fy
                        wts = (pxy0 * az, pxy0 * fz, pxy1 * az, pxy1 * fz,
                               pxy2 * az, pxy2 * fz, pxy3 * az, pxy3 * fz)
                        for c in range(8):
                            w_v[l, c, pl.ds(p, L)] = wts[c]
                        if l == 0:
                            z1 = jnp.minimum(zi + 1, r - 1)
                            corners = (a00 + zi, a00 + z1, a01 + zi, a01 + z1,
                                       a10 + zi, a10 + z1, a11 + zi, a11 + z1)
                            for c in range(8):
                                idx0_v[c, pl.ds(p, L)] = corners[c]
                        else:
                            pairs = (a00 + zi, a01 + zi, a10 + zi, a11 + zi)
                            for q in range(4):
                                pidx_v[l - 1, q, h, pl.ds(pp, L)] = pairs[q]

            copies = []
            for li, zd_hbm in ((0, zd1_hbm), (1, zd2_hbm), (2, zd3_hbm)):
                for q in range(4):
                    for h in range(NH):
                        copies.append(pltpu.async_copy(
                            zd_hbm.at[pidx_v.at[li, q, h]],
                            rows_v.at[li, q, h], sem))
            for cp in copies:
                cp.wait()

            @pl.loop(0, NH)
            def _acch(h):
                @pl.loop(0, H // L)
                def _acc(jj):
                    p = h * H + jj * L
                    row = p + iota
                    rloc = jj * L + iota
                    for f in range(F):
                        col = jnp.full((L,), f, jnp.int32)
                        acc = jnp.zeros((L,), jnp.float32)
                        for c in range(8):
                            wv = w_v[0, c, pl.ds(p, L)]
                            ridx = idx0_v[c, pl.ds(p, L)]
                            acc = acc + wv * plsc.load_gather(g0_v, [ridx, col])
                        plsc.store_scatter(
                            feats_v, [row, jnp.full((L,), f, jnp.int32)], acc)
                    for li in range(3):
                        for f in range(F):
                            col0 = jnp.full((L,), f, jnp.int32)
                            col1 = jnp.full((L,), F + f, jnp.int32)
                            acc = jnp.zeros((L,), jnp.float32)
                            for q in range(4):
                                w0 = w_v[li + 1, 2 * q, pl.ds(p, L)]
                                w1 = w_v[li + 1, 2 * q + 1, pl.ds(p, L)]
                                r_ref = rows_v.at[li, q, h]
                                acc = acc + w0 * plsc.load_gather(r_ref, [rloc, col0])
                                acc = acc + w1 * plsc.load_gather(r_ref, [rloc, col1])
                            plsc.store_scatter(
                                feats_v,
                                [row, jnp.full((L,), (li + 1) * F + f, jnp.int32)],
                                acc)

            pltpu.sync_copy(feats_v, feats_hbm.at[pl.ds(base, B)])

    kern = pl.kernel(
        body,
        out_type=jax.ShapeDtypeStruct((N_POINTS, NLEV * F), jnp.float32),
        mesh=mesh,
        compiler_params=pltpu.CompilerParams(needs_layout_passes=False,
                                             use_tc_tiling_on_sc=False),
        scratch_types=[
            pltpu.VMEM((RES[0] ** 3, F), jnp.float32),
            pltpu.VMEM((B,), jnp.float32),
            pltpu.VMEM((B,), jnp.float32),
            pltpu.VMEM((B,), jnp.float32),
            pltpu.VMEM((8, B), jnp.int32),
            pltpu.VMEM((NLEV - 1, 4, NH, H), jnp.int32),
            pltpu.VMEM((NLEV, 8, B), jnp.float32),
            pltpu.VMEM((NLEV - 1, 4, NH, H, F2), jnp.float32),
            pltpu.VMEM((B, NLEV * F), jnp.float32),
            pltpu.SemaphoreType.DMA,
        ],
    )
    return kern(xs, ys, zs, g0, zd1, zd2, zd3)


BN = 2048


def _mlp_body(x_ref, w0, b0, w1, b1, w2, b2, w3, b3, o_ref):
    x = x_ref[...]
    h = jnp.maximum(jnp.dot(x, w0[...], preferred_element_type=jnp.float32)
                    + b0[...], 0.0)
    h = jnp.maximum(jnp.dot(h, w1[...], preferred_element_type=jnp.float32)
                    + b1[...], 0.0)
    h = jnp.maximum(jnp.dot(h, w2[...], preferred_element_type=jnp.float32)
                    + b2[...], 0.0)
    o_ref[...] = jnp.dot(h, w3[...], preferred_element_type=jnp.float32) + b3[...]


def _mlp(feats, W0, b0, W1, b1, W2, b2, W3, b3):
    def wspec(shape):
        return pl.BlockSpec(shape, lambda i: (0, 0))

    return pl.pallas_call(
        _mlp_body,
        grid=(N_POINTS // BN,),
        in_specs=[pl.BlockSpec((BN, NLEV * F), lambda i: (i, 0)),
                  wspec((32, 64)), wspec((1, 64)),
                  wspec((64, 64)), wspec((1, 64)),
                  wspec((64, 64)), wspec((1, 64)),
                  wspec((64, 4)), wspec((1, 4))],
        out_specs=pl.BlockSpec((BN, 4), lambda i: (i, 0)),
        out_shape=jax.ShapeDtypeStruct((N_POINTS, 4), jnp.float32),
    )(feats, W0, b0.reshape(1, -1), W1, b1.reshape(1, -1),
      W2, b2.reshape(1, -1), W3, b3.reshape(1, -1))


def _zpair(g):
    zn = jnp.concatenate([g[:, :, 1:, :], g[:, :, -1:, :]], axis=2)
    return jnp.concatenate([g, zn], axis=-1).reshape(-1, F2)


@jax.jit
def kernel(coords, grid0, grid1, grid2, grid3, W0, b0, W1, b1, W2, b2, W3, b3):
    xs, ys, zs = coords[:, 0], coords[:, 1], coords[:, 2]
    g0 = grid0.reshape(-1, F)
    feats = _sc_trilerp_feats(xs, ys, zs, g0,
                              _zpair(grid1), _zpair(grid2), _zpair(grid3))
    return _mlp(feats, W0, b0, W1, b1, W2, b2, W3, b3)


# speedup vs baseline: 37.4430x; 1.2484x over previous
import jax
import jax.numpy as jnp
from jax import lax
from jax.experimental import pallas as pl
from jax.experimental.pallas import tpu as pltpu
from jax.experimental.pallas import tpu_sc as plsc

RES = (16, 32, 64, 128)
NLEV = 4
F = 8
F2 = 2 * F
N_POINTS = 524288
NC, NS, L = 2, 16, 16
NW = NC * NS
PTS_PER_W = N_POINTS // NW
B = 128
NBLK = PTS_PER_W // B
NSUB = B // L


def _make_sc(levels):
    has0 = 0 in levels
    glev = tuple(l for l in levels if l > 0)
    ng = len(glev)
    fdim = F * len(levels)
    mesh = plsc.VectorSubcoreMesh(core_axis_name="c", subcore_axis_name="s",
                                  num_cores=NC, num_subcores=NS)

    def body(*refs):
        it = iter(refs)
        xs_hbm, ys_hbm, zs_hbm = next(it), next(it), next(it)
        g0_hbm = next(it) if has0 else None
        zd_hbm = tuple(next(it) for _ in range(ng))
        feats_hbm = next(it)
        g0_v = next(it) if has0 else None
        xs_v, ys_v, zs_v = next(it), next(it), next(it)
        idx0_v = next(it) if has0 else None
        pidx_v = next(it) if ng else None
        w_v = next(it)
        rows_v = next(it) if ng else None
        feats_v = next(it)
        sems = (next(it), next(it))

        wid = lax.axis_index("s") * NC + lax.axis_index("c")
        w_base = wid * PTS_PER_W
        if has0:
            pltpu.sync_copy(g0_hbm, g0_v)
        iota = lax.iota(jnp.int32, L)

        def fire(par, g):
            base = w_base + g * B
            pltpu.sync_copy(xs_hbm.at[pl.ds(base, B)], xs_v.at[par])
            pltpu.sync_copy(ys_hbm.at[pl.ds(base, B)], ys_v.at[par])
            pltpu.sync_copy(zs_hbm.at[pl.ds(base, B)], zs_v.at[par])

            @pl.loop(0, NSUB)
            def _idx(j):
                p = j * L
                x = xs_v[par, pl.ds(p, L)]
                y = ys_v[par, pl.ds(p, L)]
                z = zs_v[par, pl.ds(p, L)]
                for k, l in enumerate(levels):
                    r = RES[l]
                    rm1 = float(r - 1)
                    gx = jnp.clip(x * rm1, 0.0, rm1)
                    gy = jnp.clip(y * rm1, 0.0, rm1)
                    gz = jnp.clip(z * rm1, 0.0, rm1)
                    xi = gx.astype(jnp.int32)
                    yi = gy.astype(jnp.int32)
                    zi = gz.astype(jnp.int32)
                    fx = gx - xi.astype(jnp.float32)
                    fy = gy - yi.astype(jnp.float32)
                    fz = gz - zi.astype(jnp.float32)
                    x1 = jnp.minimum(xi + 1, r - 1)
                    y1 = jnp.minimum(yi + 1, r - 1)
                    a00 = (xi * r + yi) * r
                    a01 = (xi * r + y1) * r
                    a10 = (x1 * r + yi) * r
                    a11 = (x1 * r + y1) * r
                    ax, ay, az = 1.0 - fx, 1.0 - fy, 1.0 - fz
                    pxy0, pxy1 = ax * ay, ax * fy
                    pxy2, pxy3 = fx * ay, fx * fy
                    wts = (pxy0 * az, pxy0 * fz, pxy1 * az, pxy1 * fz,
                           pxy2 * az, pxy2 * fz, pxy3 * az, pxy3 * fz)
                    for c in range(8):
                        w_v[par, k, c, pl.ds(p, L)] = wts[c]
                    if l == 0:
                        z1 = jnp.minimum(zi + 1, r - 1)
                        corners = (a00 + zi, a00 + z1, a01 + zi, a01 + z1,
                                   a10 + zi, a10 + z1, a11 + zi, a11 + z1)
                        for c in range(8):
                            idx0_v[par, c, pl.ds(p, L)] = corners[c]
                    else:
                        li = glev.index(l)
                        pairs = (a00 + zi, a01 + zi, a10 + zi, a11 + zi)
                        for q in range(4):
                            pidx_v[par, li, q, pl.ds(p, L)] = pairs[q]

            for li in range(ng):
                for q in range(4):
                    pltpu.async_copy(zd_hbm[li].at[pidx_v.at[par, li, q]],
                                     rows_v.at[par, li, q], sems[par])

        def drain(par):
            for li in range(ng):
                for q in range(4):
                    pltpu.make_async_copy(zd_hbm[li].at[pidx_v.at[par, li, q]],
                                          rows_v.at[par, li, q],
                                          sems[par]).wait()

        def compute(par, g):
            base = w_base + g * B

            @pl.loop(0, NSUB)
            def _acc(j):
                p = j * L
                row = p + iota
                fout = 0
                for k, l in enumerate(levels):
                    if l == 0:
                        for f in range(F):
                            col = jnp.full((L,), f, jnp.int32)
                            acc = jnp.zeros((L,), jnp.float32)
                            for c in range(8):
                                wv = w_v[par, k, c, pl.ds(p, L)]
                                ridx = idx0_v[par, c, pl.ds(p, L)]
                                acc = acc + wv * plsc.load_gather(
                                    g0_v, [ridx, col])
                            plsc.store_scatter(
                                feats_v,
                                [row, jnp.full((L,), fout + f, jnp.int32)], acc)
                    else:
                        li = glev.index(l)
                        for f in range(F):
                            col0 = jnp.full((L,), f, jnp.int32)
                            col1 = jnp.full((L,), F + f, jnp.int32)
                            acc = jnp.zeros((L,), jnp.float32)
                            for q in range(4):
                                w0 = w_v[par, k, 2 * q, pl.ds(p, L)]
                                w1 = w_v[par, k, 2 * q + 1, pl.ds(p, L)]
                                r_ref = rows_v.at[par, li, q]
                                acc = acc + w0 * plsc.load_gather(
                                    r_ref, [row, col0])
                                acc = acc + w1 * plsc.load_gather(
                                    r_ref, [row, col1])
                            plsc.store_scatter(
                                feats_v,
                                [row, jnp.full((L,), fout + f, jnp.int32)], acc)
                    fout += F

            pltpu.sync_copy(feats_v, feats_hbm.at[pl.ds(base, B)])

        fire(0, 0)

        @pl.loop(0, NBLK, step=2)
        def _blk(g):
            fire(1, g + 1)
            drain(0)
            compute(0, g)

            @pl.when(g + 2 < NBLK)
            def _next():
                fire(0, g + 2)

            drain(1)
            compute(1, g + 1)

    scratch = []
    if has0:
        scratch.append(pltpu.VMEM((RES[0] ** 3, F), jnp.float32))
    scratch += [pltpu.VMEM((2, B), jnp.float32)] * 3
    if has0:
        scratch.append(pltpu.VMEM((2, 8, B), jnp.int32))
    if ng:
        scratch.append(pltpu.VMEM((2, ng, 4, B), jnp.int32))
    scratch.append(pltpu.VMEM((2, len(levels), 8, B), jnp.float32))
    if ng:
        scratch.append(pltpu.VMEM((2, ng, 4, B, F2), jnp.float32))
    scratch.append(pltpu.VMEM((B, fdim), jnp.float32))
    scratch += [pltpu.SemaphoreType.DMA, pltpu.SemaphoreType.DMA]

    return pl.kernel(
        body,
        out_type=jax.ShapeDtypeStruct((N_POINTS, fdim), jnp.float32),
        mesh=mesh,
        compiler_params=pltpu.CompilerParams(needs_layout_passes=False,
                                             use_tc_tiling_on_sc=False),
        scratch_types=scratch,
    )


BN = 2048


def _mlp_body(a_ref, b_ref, w0a, w0b, b0, w1, b1, w2, b2, w3, b3, o_ref):
    x = (jnp.dot(a_ref[...], w0a[...], preferred_element_type=jnp.float32)
         + jnp.dot(b_ref[...], w0b[...], preferred_element_type=jnp.float32))
    h = jnp.maximum(x + b0[...], 0.0)
    h = jnp.maximum(jnp.dot(h, w1[...], preferred_element_type=jnp.float32)
                    + b1[...], 0.0)
    h = jnp.maximum(jnp.dot(h, w2[...], preferred_element_type=jnp.float32)
                    + b2[...], 0.0)
    o_ref[...] = jnp.dot(h, w3[...], preferred_element_type=jnp.float32) + b3[...]


def _mlp(feats_a, feats_b, W0, b0, W1, b1, W2, b2, W3, b3):
    def wspec(shape):
        return pl.BlockSpec(shape, lambda i: (0, 0))

    return pl.pallas_call(
        _mlp_body,
        grid=(N_POINTS // BN,),
        in_specs=[pl.BlockSpec((BN, 3 * F), lambda i: (i, 0)),
                  pl.BlockSpec((BN, F), lambda i: (i, 0)),
                  wspec((3 * F, 64)), wspec((F, 64)), wspec((1, 64)),
                  wspec((64, 64)), wspec((1, 64)),
                  wspec((64, 64)), wspec((1, 64)),
                  wspec((64, 4)), wspec((1, 4))],
        out_specs=pl.BlockSpec((BN, 4), lambda i: (i, 0)),
        out_shape=jax.ShapeDtypeStruct((N_POINTS, 4), jnp.float32),
    )(feats_a, feats_b, W0[:3 * F], W0[3 * F:], b0.reshape(1, -1),
      W1, b1.reshape(1, -1), W2, b2.reshape(1, -1), W3, b3.reshape(1, -1))


def _zpair(g):
    zn = jnp.concatenate([g[:, :, 1:, :], g[:, :, -1:, :]], axis=2)
    return jnp.concatenate([g, zn], axis=-1).reshape(-1, F2)


@jax.jit
def kernel(coords, grid0, grid1, grid2, grid3, W0, b0, W1, b1, W2, b2, W3, b3):
    xs, ys, zs = coords[:, 0], coords[:, 1], coords[:, 2]
    g0 = grid0.reshape(-1, F)
    feats_a = _make_sc((0, 1, 2))(xs, ys, zs, g0, _zpair(grid1), _zpair(grid2))
    feats_b = _make_sc((3,))(xs, ys, zs, _zpair(grid3))
    return _mlp(feats_a, feats_b, W0, b0, W1, b1, W2, b2, W3, b3)
